# baseline (speedup 1.0000x reference)
import functools
import math

import jax
import jax.numpy as jnp
from jax import lax
from jax.experimental import pallas as pl
from jax.experimental.pallas import tpu as pltpu


def _ln_in(x, g, b):
    m = jnp.mean(x, axis=-1, keepdims=True)
    v = jnp.var(x, axis=-1, keepdims=True)
    return (x - m) / jnp.sqrt(v + 1e-5) * g + b


def _dot(x, w):
    return jnp.dot(x, w, preferred_element_type=jnp.float32)


def _node_dense_body(vf_ref, cv_ref, un_ref, W1_ref, b1_ref, cs_ref, une_ref,
                     Wk_ref, bk_ref, Wv_ref, bv_ref, Wq2_ref, bq2_ref,
                     fv_ref, k_ref, v_ref, q2_ref, *, K):
    vf = vf_ref[...]
    cv = cv_ref[...]
    un = un_ref[...]
    ks = lax.broadcasted_iota(jnp.int32, (1, K), 1)
    oh_c = (cv == ks).astype(jnp.float32)
    oh_u = (un == ks).astype(jnp.float32)
    fv = (_dot(vf, W1_ref[...]) + b1_ref[...]
          + _dot(oh_c, cs_ref[...]) + _dot(oh_u, une_ref[...]))
    fv_ref[...] = fv
    k_ref[...] = _dot(fv, Wk_ref[...]) + bk_ref[...]
    v_ref[...] = _dot(fv, Wv_ref[...]) + bv_ref[...]
    q2_ref[...] = _dot(fv, Wq2_ref[...]) + bq2_ref[...]


def _edge_query_body(ef_ref, Wq_ref, bq_ref, q_ref):
    q_ref[...] = _dot(ef_ref[...], Wq_ref[...]) + bq_ref[...]


def _edge_stage_body(src_ref, dst_ref, bias_ref, k_ref, q_ref, v_ref,
                     H_ref, s_ref, *, C, inv_sqrt_qd):
    @pl.when(pl.program_id(0) == 0)
    def _init():
        H_ref[...] = jnp.zeros_like(H_ref)
        s_ref[...] = jnp.zeros_like(s_ref)

    def edge(i, carry):
        si = src_ref[0, 0, i]
        di = dst_ref[0, 0, i]
        krow = k_ref[pl.ds(si, 1), :]
        qrow = q_ref[pl.ds(di, 1), :]
        t = jnp.sum(krow * qrow)
        a = jnp.where(t >= 0, t, 0.01 * t) * inv_sqrt_qd + bias_ref[0, 0, i]
        e = jnp.exp(a)
        H_ref[pl.ds(di, 1), :] = H_ref[pl.ds(di, 1), :] + e * v_ref[pl.ds(si, 1), :]
        s_ref[pl.ds(di, 1), :] = s_ref[pl.ds(di, 1), :] + e
        return carry

    lax.fori_loop(0, C, edge, 0)


def _he_dense_body(H_ref, s_ref, ef_ref, g1_ref, b1_ref, Wl1_ref, bl1_ref,
                   Wl2_ref, bl2_ref, Wk2_ref, bk2_ref, Wv2_ref, bv2_ref,
                   k2_ref, v2_ref):
    h = H_ref[...] / jnp.maximum(s_ref[...], 1e-30)
    g1 = g1_ref[...]
    b1 = b1_ref[...]
    x = _ln_in(h + ef_ref[...], g1, b1)
    f = _dot(jax.nn.relu(_dot(x, Wl1_ref[...]) + bl1_ref[...]), Wl2_ref[...]) + bl2_ref[...]
    fe = _ln_in(f + x, g1, b1)
    k2_ref[...] = _dot(fe, Wk2_ref[...]) + bk2_ref[...]
    v2_ref[...] = _dot(fe, Wv2_ref[...]) + bv2_ref[...]


def _final_body(H2_ref, s2_ref, fv_ref, g2_ref, b2_ref, Wl3_ref, bl3_ref,
                Wl4_ref, bl4_ref, Wm_ref, bm_ref, out_ref):
    h2 = H2_ref[...] / jnp.maximum(s2_ref[...], 1e-30)
    g2 = g2_ref[...]
    b2 = b2_ref[...]
    y = _ln_in(h2 + fv_ref[...], g2, b2)
    f2 = _dot(jax.nn.relu(_dot(y, Wl3_ref[...]) + bl3_ref[...]), Wl4_ref[...]) + bl4_ref[...]
    fv2 = _ln_in(f2 + y, g2, b2)
    out_ref[...] = _dot(fv2, Wm_ref[...]) + bm_ref[...]


def _full(shape):
    nd = len(shape)
    return pl.BlockSpec(shape, lambda *a: (0,) * nd)


def _edge_stage(src, dst, bias, k, q, v, nseg, inv_sqrt_qd):
    E = src.shape[0]
    D = k.shape[1]
    C = 1000 if E % 1000 == 0 else E
    nb = E // C
    src3 = src.astype(jnp.int32).reshape(nb, 1, C)
    dst3 = dst.astype(jnp.int32).reshape(nb, 1, C)
    bias3 = bias.reshape(nb, 1, C)
    idx_spec = pl.BlockSpec((1, 1, C), lambda i: (i, 0, 0), memory_space=pltpu.SMEM)
    H, s = pl.pallas_call(
        functools.partial(_edge_stage_body, C=C, inv_sqrt_qd=inv_sqrt_qd),
        grid=(nb,),
        in_specs=[idx_spec, idx_spec, idx_spec,
                  _full(k.shape), _full(q.shape), _full(v.shape)],
        out_specs=[_full((nseg, D)), _full((nseg, 1))],
        out_shape=[jax.ShapeDtypeStruct((nseg, D), jnp.float32),
                   jax.ShapeDtypeStruct((nseg, 1), jnp.float32)],
    )(src3, dst3, bias3, k, q, v)
    return H, s


def kernel(vfeat, efeat, bias_in, bias_con, W_vtx1, b_vtx1, cs_emb, un_emb,
           W_kv, b_kv, W_vv, b_vv, W_qe, b_qe, W_ke, b_ke, W_ve, b_ve,
           W_qv, b_qv, ln1_g, ln1_b, ln2_g, ln2_b, W_l1, b_l1, W_l2, b_l2,
           W_l3, b_l3, W_l4, b_l4, W_mlp, b_mlp, centrality_values,
           uniqueness, in_src, in_dst, con_src, con_dst):
    N, D = vfeat.shape
    M = efeat.shape[0]
    K = cs_emb.shape[0]
    qd = W_kv.shape[1]
    inv_sqrt_qd = 1.0 / math.sqrt(qd)
    Dm = W_mlp.shape[1]

    r = lambda x: x.reshape(1, -1)
    cv2 = centrality_values.astype(jnp.int32).reshape(N, 1)
    un2 = uniqueness.astype(jnp.int32).reshape(N, 1)

    Bn = 2000 if N % 2000 == 0 else N
    row = lambda w: pl.BlockSpec((Bn, w), lambda i: (i, 0))
    fv, k, v, q2 = pl.pallas_call(
        functools.partial(_node_dense_body, K=K),
        grid=(N // Bn,),
        in_specs=[row(D), pl.BlockSpec((Bn, 1), lambda i: (i, 0)),
                  pl.BlockSpec((Bn, 1), lambda i: (i, 0)),
                  _full((D, D)), _full((1, D)), _full((K, D)), _full((K, D)),
                  _full((D, D)), _full((1, D)), _full((D, D)), _full((1, D)),
                  _full((D, D)), _full((1, D))],
        out_specs=[row(D), row(D), row(D), row(D)],
        out_shape=[jax.ShapeDtypeStruct((N, D), jnp.float32)] * 4,
    )(vfeat, cv2, un2, W_vtx1, r(b_vtx1), cs_emb, un_emb,
      W_kv, r(b_kv), W_vv, r(b_vv), W_qv, r(b_qv))

    q = pl.pallas_call(
        _edge_query_body,
        in_specs=[_full((M, D)), _full((D, D)), _full((1, D))],
        out_specs=_full((M, D)),
        out_shape=jax.ShapeDtypeStruct((M, D), jnp.float32),
    )(efeat, W_qe, r(b_qe))

    H, s = _edge_stage(in_src, in_dst, bias_in, k, q, v, M, inv_sqrt_qd)

    k2, v2 = pl.pallas_call(
        _he_dense_body,
        in_specs=[_full((M, D)), _full((M, 1)), _full((M, D)),
                  _full((1, D)), _full((1, D)),
                  _full((D, D)), _full((1, D)), _full((D, D)), _full((1, D)),
                  _full((D, D)), _full((1, D)), _full((D, D)), _full((1, D))],
        out_specs=[_full((M, D)), _full((M, D))],
        out_shape=[jax.ShapeDtypeStruct((M, D), jnp.float32)] * 2,
    )(H, s, efeat, r(ln1_g), r(ln1_b), W_l1, r(b_l1), W_l2, r(b_l2),
      W_ke, r(b_ke), W_ve, r(b_ve))

    H2, s2 = _edge_stage(con_src, con_dst, bias_con, k2, q2, v2, N, inv_sqrt_qd)

    out = pl.pallas_call(
        _final_body,
        grid=(N // Bn,),
        in_specs=[row(D), pl.BlockSpec((Bn, 1), lambda i: (i, 0)), row(D),
                  _full((1, D)), _full((1, D)),
                  _full((D, D)), _full((1, D)), _full((D, D)), _full((1, D)),
                  _full((D, Dm)), _full((1, Dm))],
        out_specs=row(Dm),
        out_shape=jax.ShapeDtypeStruct((N, Dm), jnp.float32),
    )(H2, s2, fv, r(ln2_g), r(ln2_b), W_l3, r(b_l3), W_l4, r(b_l4),
      W_mlp, r(b_mlp))
    return out


# speedup vs baseline: 7.3511x; 7.3511x over previous
import functools
import math

import jax
import jax.numpy as jnp
from jax import lax
from jax.experimental import pallas as pl
from jax.experimental.pallas import tpu as pltpu


def _ln_in(x, g, b):
    m = jnp.mean(x, axis=-1, keepdims=True)
    v = jnp.var(x, axis=-1, keepdims=True)
    return (x - m) / jnp.sqrt(v + 1e-5) * g + b


def _dot(x, w):
    return jnp.dot(x, w, preferred_element_type=jnp.float32)


def _node_dense_body(vf_ref, cv_ref, un_ref, W1_ref, b1_ref, cs_ref, une_ref,
                     Wk_ref, bk_ref, Wv_ref, bv_ref, Wq2_ref, bq2_ref,
                     fv_ref, k_ref, v_ref, q2_ref, *, K):
    vf = vf_ref[...]
    cv = cv_ref[...]
    un = un_ref[...]
    ks = lax.broadcasted_iota(jnp.int32, (1, K), 1)
    oh_c = (cv == ks).astype(jnp.float32)
    oh_u = (un == ks).astype(jnp.float32)
    fv = (_dot(vf, W1_ref[...]) + b1_ref[...]
          + _dot(oh_c, cs_ref[...]) + _dot(oh_u, une_ref[...]))
    fv_ref[...] = fv
    k_ref[...] = _dot(fv, Wk_ref[...]) + bk_ref[...]
    vv = _dot(fv, Wv_ref[...]) + bv_ref[...]
    v_ref[...] = jnp.concatenate(
        [vv, jnp.ones((vv.shape[0], 128), jnp.float32)], axis=1)
    q2_ref[...] = _dot(fv, Wq2_ref[...]) + bq2_ref[...]


def _edge_query_body(ef_ref, Wq_ref, bq_ref, q_ref):
    q_ref[...] = _dot(ef_ref[...], Wq_ref[...]) + bq_ref[...]


def _edge_stage_body(src_ref, dst_ref, bias_ref, k_ref, q_ref, v_ref,
                     H_ref, *, C, inv_sqrt_qd, GB=8):
    @pl.when(pl.program_id(0) == 0)
    def _init():
        H_ref[...] = jnp.zeros_like(H_ref)

    def group(g, carry):
        base = g * GB
        idxs = [(src_ref[0, 0, base + j], dst_ref[0, 0, base + j])
                for j in range(GB)]
        krows = jnp.concatenate([k_ref[pl.ds(s, 1), :] for s, _ in idxs], 0)
        qrows = jnp.concatenate([q_ref[pl.ds(d, 1), :] for _, d in idxs], 0)
        t = jnp.sum(krows * qrows, axis=1, keepdims=True)
        b = jnp.concatenate(
            [bias_ref[0, 0, base + j].reshape(1, 1) for j in range(GB)], 0)
        e = jnp.exp(jnp.where(t >= 0, t, 0.01 * t) * inv_sqrt_qd + b)
        for j, (s, d) in enumerate(idxs):
            H_ref[pl.ds(d, 1), :] = (H_ref[pl.ds(d, 1), :]
                                     + e[j, 0] * v_ref[pl.ds(s, 1), :])
        return carry

    lax.fori_loop(0, C // GB, group, 0)


def _he_dense_body(H_ref, ef_ref, g1_ref, b1_ref, Wl1_ref, bl1_ref,
                   Wl2_ref, bl2_ref, Wk2_ref, bk2_ref, Wv2_ref, bv2_ref,
                   k2_ref, v2_ref, *, D):
    Hfull = H_ref[...]
    h = Hfull[:, :D] / jnp.maximum(Hfull[:, D:D + 1], 1e-30)
    g1 = g1_ref[...]
    b1 = b1_ref[...]
    x = _ln_in(h + ef_ref[...], g1, b1)
    f = _dot(jax.nn.relu(_dot(x, Wl1_ref[...]) + bl1_ref[...]), Wl2_ref[...]) + bl2_ref[...]
    fe = _ln_in(f + x, g1, b1)
    k2_ref[...] = _dot(fe, Wk2_ref[...]) + bk2_ref[...]
    v2 = _dot(fe, Wv2_ref[...]) + bv2_ref[...]
    v2_ref[...] = jnp.concatenate(
        [v2, jnp.ones((v2.shape[0], 128), jnp.float32)], axis=1)


def _final_body(H2_ref, fv_ref, g2_ref, b2_ref, Wl3_ref, bl3_ref,
                Wl4_ref, bl4_ref, Wm_ref, bm_ref, out_ref, *, D):
    H2full = H2_ref[...]
    h2 = H2full[:, :D] / jnp.maximum(H2full[:, D:D + 1], 1e-30)
    g2 = g2_ref[...]
    b2 = b2_ref[...]
    y = _ln_in(h2 + fv_ref[...], g2, b2)
    f2 = _dot(jax.nn.relu(_dot(y, Wl3_ref[...]) + bl3_ref[...]), Wl4_ref[...]) + bl4_ref[...]
    fv2 = _ln_in(f2 + y, g2, b2)
    out_ref[...] = _dot(fv2, Wm_ref[...]) + bm_ref[...]


def _full(shape):
    nd = len(shape)
    return pl.BlockSpec(shape, lambda *a: (0,) * nd)


def _edge_stage(src, dst, bias, k, q, v, nseg, inv_sqrt_qd):
    E = src.shape[0]
    Dx = v.shape[1]
    C = 1000 if E % 1000 == 0 else E
    nb = E // C
    src3 = src.astype(jnp.int32).reshape(nb, 1, C)
    dst3 = dst.astype(jnp.int32).reshape(nb, 1, C)
    bias3 = bias.reshape(nb, 1, C)
    idx_spec = pl.BlockSpec((1, 1, C), lambda i: (i, 0, 0), memory_space=pltpu.SMEM)
    H = pl.pallas_call(
        functools.partial(_edge_stage_body, C=C, inv_sqrt_qd=inv_sqrt_qd),
        grid=(nb,),
        in_specs=[idx_spec, idx_spec, idx_spec,
                  _full(k.shape), _full(q.shape), _full(v.shape)],
        out_specs=_full((nseg, Dx)),
        out_shape=jax.ShapeDtypeStruct((nseg, Dx), jnp.float32),
    )(src3, dst3, bias3, k, q, v)
    return H


def kernel(vfeat, efeat, bias_in, bias_con, W_vtx1, b_vtx1, cs_emb, un_emb,
           W_kv, b_kv, W_vv, b_vv, W_qe, b_qe, W_ke, b_ke, W_ve, b_ve,
           W_qv, b_qv, ln1_g, ln1_b, ln2_g, ln2_b, W_l1, b_l1, W_l2, b_l2,
           W_l3, b_l3, W_l4, b_l4, W_mlp, b_mlp, centrality_values,
           uniqueness, in_src, in_dst, con_src, con_dst):
    N, D = vfeat.shape
    M = efeat.shape[0]
    K = cs_emb.shape[0]
    qd = W_kv.shape[1]
    inv_sqrt_qd = 1.0 / math.sqrt(qd)
    Dm = W_mlp.shape[1]

    r = lambda x: x.reshape(1, -1)
    cv2 = centrality_values.astype(jnp.int32).reshape(N, 1)
    un2 = uniqueness.astype(jnp.int32).reshape(N, 1)

    Bn = 2000 if N % 2000 == 0 else N
    Dx = D + 128
    row = lambda w: pl.BlockSpec((Bn, w), lambda i: (i, 0))
    fv, k, v, q2 = pl.pallas_call(
        functools.partial(_node_dense_body, K=K),
        grid=(N // Bn,),
        in_specs=[row(D), pl.BlockSpec((Bn, 1), lambda i: (i, 0)),
                  pl.BlockSpec((Bn, 1), lambda i: (i, 0)),
                  _full((D, D)), _full((1, D)), _full((K, D)), _full((K, D)),
                  _full((D, D)), _full((1, D)), _full((D, D)), _full((1, D)),
                  _full((D, D)), _full((1, D))],
        out_specs=[row(D), row(D), row(Dx), row(D)],
        out_shape=[jax.ShapeDtypeStruct((N, D), jnp.float32),
                   jax.ShapeDtypeStruct((N, D), jnp.float32),
                   jax.ShapeDtypeStruct((N, Dx), jnp.float32),
                   jax.ShapeDtypeStruct((N, D), jnp.float32)],
    )(vfeat, cv2, un2, W_vtx1, r(b_vtx1), cs_emb, un_emb,
      W_kv, r(b_kv), W_vv, r(b_vv), W_qv, r(b_qv))

    q = pl.pallas_call(
        _edge_query_body,
        in_specs=[_full((M, D)), _full((D, D)), _full((1, D))],
        out_specs=_full((M, D)),
        out_shape=jax.ShapeDtypeStruct((M, D), jnp.float32),
    )(efeat, W_qe, r(b_qe))

    H = _edge_stage(in_src, in_dst, bias_in, k, q, v, M, inv_sqrt_qd)

    k2, v2 = pl.pallas_call(
        functools.partial(_he_dense_body, D=D),
        in_specs=[_full((M, Dx)), _full((M, D)),
                  _full((1, D)), _full((1, D)),
                  _full((D, D)), _full((1, D)), _full((D, D)), _full((1, D)),
                  _full((D, D)), _full((1, D)), _full((D, D)), _full((1, D))],
        out_specs=[_full((M, D)), _full((M, Dx))],
        out_shape=[jax.ShapeDtypeStruct((M, D), jnp.float32),
                   jax.ShapeDtypeStruct((M, Dx), jnp.float32)],
    )(H, efeat, r(ln1_g), r(ln1_b), W_l1, r(b_l1), W_l2, r(b_l2),
      W_ke, r(b_ke), W_ve, r(b_ve))

    H2 = _edge_stage(con_src, con_dst, bias_con, k2, q2, v2, N, inv_sqrt_qd)

    out = pl.pallas_call(
        functools.partial(_final_body, D=D),
        grid=(N // Bn,),
        in_specs=[row(Dx), row(D),
                  _full((1, D)), _full((1, D)),
                  _full((D, D)), _full((1, D)), _full((D, D)), _full((1, D)),
                  _full((D, Dm)), _full((1, Dm))],
        out_specs=row(Dm),
        out_shape=jax.ShapeDtypeStruct((N, Dm), jnp.float32),
    )(H2, fv, r(ln2_g), r(ln2_b), W_l3, r(b_l3), W_l4, r(b_l4),
      W_mlp, r(b_mlp))
    return out


# speedup vs baseline: 11.5542x; 1.5718x over previous
import functools
import math

import jax
import jax.numpy as jnp
from jax import lax
from jax.experimental import pallas as pl
from jax.experimental.pallas import tpu as pltpu


def _ln_in(x, g, b):
    m = jnp.mean(x, axis=-1, keepdims=True)
    v = jnp.var(x, axis=-1, keepdims=True)
    return (x - m) / jnp.sqrt(v + 1e-5) * g + b


def _dot(x, w):
    return jnp.dot(x, w, preferred_element_type=jnp.float32)


def _node_dense_body(vf_ref, cv_ref, un_ref, W1_ref, b1_ref, cs_ref, une_ref,
                     Wk_ref, bk_ref, Wv_ref, bv_ref, Wq2_ref, bq2_ref,
                     fv_ref, k_ref, v_ref, q2_ref, *, K):
    vf = vf_ref[...]
    cv = cv_ref[...]
    un = un_ref[...]
    ks = lax.broadcasted_iota(jnp.int32, (1, K), 1)
    oh_c = (cv == ks).astype(jnp.float32)
    oh_u = (un == ks).astype(jnp.float32)
    fv = (_dot(vf, W1_ref[...]) + b1_ref[...]
          + _dot(oh_c, cs_ref[...]) + _dot(oh_u, une_ref[...]))
    fv_ref[...] = fv
    k_ref[...] = _dot(fv, Wk_ref[...]) + bk_ref[...]
    vv = _dot(fv, Wv_ref[...]) + bv_ref[...]
    v_ref[...] = jnp.concatenate(
        [vv, jnp.ones((vv.shape[0], 128), jnp.float32)], axis=1)
    q2_ref[...] = _dot(fv, Wq2_ref[...]) + bq2_ref[...]


def _edge_query_body(ef_ref, Wq_ref, bq_ref, q_ref):
    q_ref[...] = _dot(ef_ref[...], Wq_ref[...]) + bq_ref[...]


def _edge_stage_body(src_ref, dst_ref, bias_ref, k_ref, q_ref, v_ref,
                     H_ref, *, C, inv_sqrt_qd, GB=16):
    @pl.when(pl.program_id(0) == 0)
    def _init():
        H_ref[...] = jnp.zeros_like(H_ref)

    def group(g, carry):
        base = g * GB
        idxs = [(src_ref[0, 0, base + j], dst_ref[0, 0, base + j])
                for j in range(GB)]
        krows = jnp.concatenate([k_ref[pl.ds(s, 1), :] for s, _ in idxs], 0)
        qrows = jnp.concatenate([q_ref[pl.ds(d, 1), :] for _, d in idxs], 0)
        t = jnp.sum(krows * qrows, axis=1, keepdims=True)
        b = jnp.concatenate(
            [bias_ref[0, 0, base + j].reshape(1, 1) for j in range(GB)], 0)
        e = jnp.exp(jnp.where(t >= 0, t, 0.01 * t) * inv_sqrt_qd + b)
        for j, (s, d) in enumerate(idxs):
            H_ref[pl.ds(d, 1), :] = (H_ref[pl.ds(d, 1), :]
                                     + e[j, 0] * v_ref[pl.ds(s, 1), :])
        return carry

    lax.fori_loop(0, C // GB, group, 0)


def _he_dense_body(H_ref, ef_ref, g1_ref, b1_ref, Wl1_ref, bl1_ref,
                   Wl2_ref, bl2_ref, Wk2_ref, bk2_ref, Wv2_ref, bv2_ref,
                   k2_ref, v2_ref, *, D):
    Hfull = H_ref[...]
    h = Hfull[:, :D] / jnp.maximum(Hfull[:, D:D + 1], 1e-30)
    g1 = g1_ref[...]
    b1 = b1_ref[...]
    x = _ln_in(h + ef_ref[...], g1, b1)
    f = _dot(jax.nn.relu(_dot(x, Wl1_ref[...]) + bl1_ref[...]), Wl2_ref[...]) + bl2_ref[...]
    fe = _ln_in(f + x, g1, b1)
    k2_ref[...] = _dot(fe, Wk2_ref[...]) + bk2_ref[...]
    v2 = _dot(fe, Wv2_ref[...]) + bv2_ref[...]
    v2_ref[...] = jnp.concatenate(
        [v2, jnp.ones((v2.shape[0], 128), jnp.float32)], axis=1)


def _final_body(H2_ref, fv_ref, g2_ref, b2_ref, Wl3_ref, bl3_ref,
                Wl4_ref, bl4_ref, Wm_ref, bm_ref, out_ref, *, D):
    H2full = H2_ref[...]
    h2 = H2full[:, :D] / jnp.maximum(H2full[:, D:D + 1], 1e-30)
    g2 = g2_ref[...]
    b2 = b2_ref[...]
    y = _ln_in(h2 + fv_ref[...], g2, b2)
    f2 = _dot(jax.nn.relu(_dot(y, Wl3_ref[...]) + bl3_ref[...]), Wl4_ref[...]) + bl4_ref[...]
    fv2 = _ln_in(f2 + y, g2, b2)
    out_ref[...] = _dot(fv2, Wm_ref[...]) + bm_ref[...]


def _full(shape):
    nd = len(shape)
    return pl.BlockSpec(shape, lambda *a: (0,) * nd)


def _edge_stage(src, dst, bias, k, q, v, nseg, inv_sqrt_qd):
    E = src.shape[0]
    Dx = v.shape[1]
    C = 800 if E % 800 == 0 else E
    nb = E // C
    src3 = src.astype(jnp.int32).reshape(nb, 1, C)
    dst3 = dst.astype(jnp.int32).reshape(nb, 1, C)
    bias3 = bias.reshape(nb, 1, C)
    idx_spec = pl.BlockSpec((1, 1, C), lambda i: (i, 0, 0), memory_space=pltpu.SMEM)
    H = pl.pallas_call(
        functools.partial(_edge_stage_body, C=C, inv_sqrt_qd=inv_sqrt_qd),
        grid=(nb,),
        in_specs=[idx_spec, idx_spec, idx_spec,
                  _full(k.shape), _full(q.shape), _full(v.shape)],
        out_specs=_full((nseg, Dx)),
        out_shape=jax.ShapeDtypeStruct((nseg, Dx), jnp.float32),
    )(src3, dst3, bias3, k, q, v)
    return H


def kernel(vfeat, efeat, bias_in, bias_con, W_vtx1, b_vtx1, cs_emb, un_emb,
           W_kv, b_kv, W_vv, b_vv, W_qe, b_qe, W_ke, b_ke, W_ve, b_ve,
           W_qv, b_qv, ln1_g, ln1_b, ln2_g, ln2_b, W_l1, b_l1, W_l2, b_l2,
           W_l3, b_l3, W_l4, b_l4, W_mlp, b_mlp, centrality_values,
           uniqueness, in_src, in_dst, con_src, con_dst):
    N, D = vfeat.shape
    M = efeat.shape[0]
    K = cs_emb.shape[0]
    qd = W_kv.shape[1]
    inv_sqrt_qd = 1.0 / math.sqrt(qd)
    Dm = W_mlp.shape[1]

    r = lambda x: x.reshape(1, -1)
    cv2 = centrality_values.astype(jnp.int32).reshape(N, 1)
    un2 = uniqueness.astype(jnp.int32).reshape(N, 1)

    Bn = 2000 if N % 2000 == 0 else N
    Dx = D + 128
    row = lambda w: pl.BlockSpec((Bn, w), lambda i: (i, 0))
    fv, k, v, q2 = pl.pallas_call(
        functools.partial(_node_dense_body, K=K),
        grid=(N // Bn,),
        in_specs=[row(D), pl.BlockSpec((Bn, 1), lambda i: (i, 0)),
                  pl.BlockSpec((Bn, 1), lambda i: (i, 0)),
                  _full((D, D)), _full((1, D)), _full((K, D)), _full((K, D)),
                  _full((D, D)), _full((1, D)), _full((D, D)), _full((1, D)),
                  _full((D, D)), _full((1, D))],
        out_specs=[row(D), row(D), row(Dx), row(D)],
        out_shape=[jax.ShapeDtypeStruct((N, D), jnp.float32),
                   jax.ShapeDtypeStruct((N, D), jnp.float32),
                   jax.ShapeDtypeStruct((N, Dx), jnp.float32),
                   jax.ShapeDtypeStruct((N, D), jnp.float32)],
    )(vfeat, cv2, un2, W_vtx1, r(b_vtx1), cs_emb, un_emb,
      W_kv, r(b_kv), W_vv, r(b_vv), W_qv, r(b_qv))

    q = pl.pallas_call(
        _edge_query_body,
        in_specs=[_full((M, D)), _full((D, D)), _full((1, D))],
        out_specs=_full((M, D)),
        out_shape=jax.ShapeDtypeStruct((M, D), jnp.float32),
    )(efeat, W_qe, r(b_qe))

    H = _edge_stage(in_src, in_dst, bias_in, k, q, v, M, inv_sqrt_qd)

    k2, v2 = pl.pallas_call(
        functools.partial(_he_dense_body, D=D),
        in_specs=[_full((M, Dx)), _full((M, D)),
                  _full((1, D)), _full((1, D)),
                  _full((D, D)), _full((1, D)), _full((D, D)), _full((1, D)),
                  _full((D, D)), _full((1, D)), _full((D, D)), _full((1, D))],
        out_specs=[_full((M, D)), _full((M, Dx))],
        out_shape=[jax.ShapeDtypeStruct((M, D), jnp.float32),
                   jax.ShapeDtypeStruct((M, Dx), jnp.float32)],
    )(H, efeat, r(ln1_g), r(ln1_b), W_l1, r(b_l1), W_l2, r(b_l2),
      W_ke, r(b_ke), W_ve, r(b_ve))

    H2 = _edge_stage(con_src, con_dst, bias_con, k2, q2, v2, N, inv_sqrt_qd)

    out = pl.pallas_call(
        functools.partial(_final_body, D=D),
        grid=(N // Bn,),
        in_specs=[row(Dx), row(D),
                  _full((1, D)), _full((1, D)),
                  _full((D, D)), _full((1, D)), _full((D, D)), _full((1, D)),
                  _full((D, Dm)), _full((1, Dm))],
        out_specs=row(Dm),
        out_shape=jax.ShapeDtypeStruct((N, Dm), jnp.float32),
    )(H2, fv, r(ln2_g), r(ln2_b), W_l3, r(b_l3), W_l4, r(b_l4),
      W_mlp, r(b_mlp))
    return out
